# baseline (speedup 1.0000x reference)
import functools

import jax
import jax.numpy as jnp
from jax import lax
from jax.experimental import pallas as pl
from jax.experimental.pallas import tpu as pltpu
from jax.experimental.pallas import tpu_sc as plsc

_HIDDEN = 2048
_BATCH = 16384
_NC = 2
_NS = 16
_NW = _NC * _NS
_ROWS_PER_W = _BATCH // _NW
_RA = 528
_RB = _BATCH // _NS - _RA

_mesh = plsc.VectorSubcoreMesh(core_axis_name="c", subcore_axis_name="s")


@functools.partial(
    pl.kernel,
    out_type=jax.ShapeDtypeStruct((_BATCH, _HIDDEN), jnp.float32),
    mesh=_mesh,
    scratch_types=[
        pltpu.VMEM((1, _HIDDEN), jnp.float32),
        pltpu.SemaphoreType.DMA,
    ],
)
def _broadcast_row(w_hbm, out_hbm, buf, sem):
    c = lax.axis_index("c")
    s = lax.axis_index("s")
    rows = jnp.where(c == 0, _RA, _RB)
    base = jnp.where(c == 0, s * _RA, _NS * _RA + s * _RB)
    pltpu.sync_copy(w_hbm, buf)

    def _issue_block(i, carry):
        for j in range(4):
            pltpu.async_copy(
                buf, out_hbm.at[pl.ds(base + i * 4 + j, 1)], sem
            )
        return carry

    lax.fori_loop(0, rows // 4, _issue_block, 0)
    pltpu.make_async_copy(
        out_hbm.at[pl.ds(base, _RB)],
        out_hbm.at[pl.ds(base, _RB)],
        sem,
    ).wait()
    extra = rows - _RB

    @pl.when(extra > 0)
    def _():
        pltpu.make_async_copy(
            out_hbm.at[pl.ds(base, _RA - _RB)],
            out_hbm.at[pl.ds(base, _RA - _RB)],
            sem,
        ).wait()


def kernel(x, embedding_weight):
    del x
    return _broadcast_row(embedding_weight)


# speedup vs baseline: 1.0280x; 1.0280x over previous
import functools

import jax
import jax.numpy as jnp
from jax import lax
from jax.experimental import pallas as pl
from jax.experimental.pallas import tpu as pltpu
from jax.experimental.pallas import tpu_sc as plsc

_HIDDEN = 2048
_BATCH = 16384
_NC = 2
_NS = 16
_NW = _NC * _NS
_ROWS_PER_W = _BATCH // _NW
_RA = 496
_RB = _BATCH // _NS - _RA

_mesh = plsc.VectorSubcoreMesh(core_axis_name="c", subcore_axis_name="s")


@functools.partial(
    pl.kernel,
    out_type=jax.ShapeDtypeStruct((_BATCH, _HIDDEN), jnp.float32),
    mesh=_mesh,
    scratch_types=[
        pltpu.VMEM((1, _HIDDEN), jnp.float32),
        pltpu.SemaphoreType.DMA,
    ],
)
def _broadcast_row(w_hbm, out_hbm, buf, sem):
    c = lax.axis_index("c")
    s = lax.axis_index("s")
    rows = jnp.where(c == 0, _RA, _RB)
    base = jnp.where(c == 0, s * _RA, _NS * _RA + s * _RB)
    pltpu.sync_copy(w_hbm, buf)

    def _issue_block(i, carry):
        for j in range(4):
            pltpu.async_copy(
                buf, out_hbm.at[pl.ds(base + i * 4 + j, 1)], sem
            )
        return carry

    lax.fori_loop(0, rows // 4, _issue_block, 0)

    @pl.when(c == 0)
    def _():
        pltpu.make_async_copy(
            out_hbm.at[pl.ds(base, _RA)],
            out_hbm.at[pl.ds(base, _RA)],
            sem,
        ).wait()

    @pl.when(c == 1)
    def _():
        pltpu.make_async_copy(
            out_hbm.at[pl.ds(base, _RB)],
            out_hbm.at[pl.ds(base, _RB)],
            sem,
        ).wait()


def kernel(x, embedding_weight):
    del x
    return _broadcast_row(embedding_weight)


# speedup vs baseline: 1.0896x; 1.0599x over previous
import functools

import jax
import jax.numpy as jnp
from jax import lax
from jax.experimental import pallas as pl
from jax.experimental.pallas import tpu as pltpu
from jax.experimental.pallas import tpu_sc as plsc

_HIDDEN = 2048
_BATCH = 16384
_NC = 2
_NS = 16
_NW = _NC * _NS
_ROWS_PER_W = _BATCH // _NW
_RA = 480
_RB = _BATCH // _NS - _RA

_mesh = plsc.VectorSubcoreMesh(core_axis_name="c", subcore_axis_name="s")


@functools.partial(
    pl.kernel,
    out_type=jax.ShapeDtypeStruct((_BATCH, _HIDDEN), jnp.float32),
    mesh=_mesh,
    scratch_types=[
        pltpu.VMEM((1, _HIDDEN), jnp.float32),
        pltpu.SemaphoreType.DMA,
    ],
)
def _broadcast_row(w_hbm, out_hbm, buf, sem):
    c = lax.axis_index("c")
    s = lax.axis_index("s")
    rows = jnp.where(c == 0, _RA, _RB)
    base = jnp.where(c == 0, s * _RA, _NS * _RA + s * _RB)
    pltpu.sync_copy(w_hbm, buf)

    def _issue_block(i, carry):
        for j in range(4):
            pltpu.async_copy(
                buf, out_hbm.at[pl.ds(base + i * 4 + j, 1)], sem
            )
        return carry

    lax.fori_loop(0, rows // 4, _issue_block, 0)

    @pl.when(c == 0)
    def _():
        pltpu.make_async_copy(
            out_hbm.at[pl.ds(base, _RA)],
            out_hbm.at[pl.ds(base, _RA)],
            sem,
        ).wait()

    @pl.when(c == 1)
    def _():
        pltpu.make_async_copy(
            out_hbm.at[pl.ds(base, _RB)],
            out_hbm.at[pl.ds(base, _RB)],
            sem,
        ).wait()


def kernel(x, embedding_weight):
    del x
    return _broadcast_row(embedding_weight)
